# baseline (speedup 1.0000x reference)
import jax
import jax.numpy as jnp
from jax.experimental import pallas as pl
from jax.experimental.pallas import tpu as pltpu

N = 16384
D = 128
BLK = 8192
NB = N // BLK


def _affine_body(obs_ref, coefs_ref, offsets_ref, z_ref, o_ref):
    ob = obs_ref[0, 0, :]
    c = jnp.zeros((BLK,), jnp.float32)
    o = jnp.zeros((BLK,), jnp.float32)
    for k in range(8):
        sel = ob == k
        c = jnp.where(sel, coefs_ref[k], c)
        o = jnp.where(sel, offsets_ref[k], o)
    o_ref[...] = c[:, None] * z_ref[...] + o[:, None]


def kernel(z, obs, coefs, offsets):
    obs3 = obs.astype(jnp.int32).reshape(NB, 1, BLK)
    return pl.pallas_call(
        _affine_body,
        grid=(NB,),
        in_specs=[
            pl.BlockSpec((1, 1, BLK), lambda i: (i, 0, 0)),
            pl.BlockSpec(memory_space=pltpu.SMEM),
            pl.BlockSpec(memory_space=pltpu.SMEM),
            pl.BlockSpec((BLK, D), lambda i: (i, 0)),
        ],
        out_specs=pl.BlockSpec((BLK, D), lambda i: (i, 0)),
        out_shape=jax.ShapeDtypeStruct((N, D), jnp.float32),
    )(obs3, coefs, offsets, z)
